# baseline (speedup 1.0000x reference)
import functools

import jax
import jax.numpy as jnp
from jax import lax
from jax.experimental import pallas as pl
from jax.experimental.pallas import tpu as pltpu
from jax.experimental.pallas import tpu_sc as plsc


def _argmin_body(x_ref, w_ref, idx_ref):
    x = x_ref[0]
    w = w_ref[...]
    K = w.shape[0]

    s2 = jax.lax.dot_general(
        -2.0 * w, x, (((1,), (0,)), ((), ())),
        preferred_element_type=jnp.float32)
    wn = jnp.sum(w * w, axis=1, keepdims=True)
    xn = jnp.sum(x * x, axis=0, keepdims=True)
    dist = (wn + xn) + s2

    fiota = jax.lax.broadcasted_iota(jnp.int32, (K, 1), 0).astype(jnp.float32)
    m = jnp.min(dist, axis=0, keepdims=True)
    fidx = jnp.min(jnp.where(dist == m, fiota, float(K)), axis=0,
                   keepdims=True)
    idx_ref[0] = fidx.astype(jnp.int32)


def _tc_indices(inputs, W):
    B, D, T = inputs.shape
    K = W.shape[0]
    idx = pl.pallas_call(
        _argmin_body,
        grid=(B,),
        in_specs=[
            pl.BlockSpec((1, D, T), lambda b: (b, 0, 0)),
            pl.BlockSpec((K, D), lambda b: (0, 0)),
        ],
        out_specs=pl.BlockSpec((1, 1, T), lambda b: (b, 0, 0)),
        out_shape=jax.ShapeDtypeStruct((B, 1, T), jnp.int32),
    )(inputs, W)
    return idx.reshape(B * T)


_DPAD = 128


def _sc_gather_rows(W, idx_flat):
    K, D = W.shape
    N = idx_flat.shape[0]
    Wp = jnp.pad(W, ((0, 0), (0, _DPAD - D)))
    info = plsc.get_sparse_core_info()
    NC, NS = info.num_cores, info.num_subcores
    NW = NC * NS
    n_per_w = N // NW
    chunk = n_per_w // 2

    @functools.partial(
        pl.kernel,
        mesh=plsc.VectorSubcoreMesh(core_axis_name="c", subcore_axis_name="s"),
        out_type=jax.ShapeDtypeStruct((N, _DPAD), jnp.float32),
        scratch_types=[
            pltpu.VMEM((chunk,), jnp.int32),
            pltpu.VMEM((chunk, _DPAD), jnp.float32),
            pltpu.SemaphoreType.DMA,
        ],
    )
    def k(w_hbm, idx_hbm, out_hbm, idx_v, rows_v, sem):
        wid = lax.axis_index("s") * NC + lax.axis_index("c")
        for h in range(2):
            base = wid * n_per_w + h * chunk
            pltpu.sync_copy(idx_hbm.at[pl.ds(base, chunk)], idx_v)
            pltpu.async_copy(w_hbm.at[idx_v], rows_v, sem).wait()
            pltpu.sync_copy(rows_v, out_hbm.at[pl.ds(base, chunk)])

    return k(Wp, idx_flat)


def _transpose_body(q_ref, out_ref):
    D = out_ref.shape[1]
    qv = q_ref[0]
    row = jax.lax.broadcasted_iota(jnp.int32, (D, _DPAD), 0)
    col = jax.lax.broadcasted_iota(jnp.int32, (D, _DPAD), 1)
    ey = (row == col).astype(jnp.float32)
    out_ref[0] = jax.lax.dot_general(
        ey, qv, (((1,), (1,)), ((), ())),
        preferred_element_type=jnp.float32)


def _tc_transpose(q_rows, B, D, T):
    return pl.pallas_call(
        _transpose_body,
        grid=(B,),
        in_specs=[pl.BlockSpec((1, T, _DPAD), lambda b: (b, 0, 0))],
        out_specs=pl.BlockSpec((1, D, T), lambda b: (b, 0, 0)),
        out_shape=jax.ShapeDtypeStruct((B, D, T), jnp.float32),
    )(q_rows.reshape(B, T, _DPAD))


def kernel(inputs, W):
    B, D, T = inputs.shape
    idx_flat = _tc_indices(inputs, W)
    q_rows = _sc_gather_rows(W, idx_flat)
    q = _tc_transpose(q_rows, B, D, T)
    return (q, idx_flat.reshape(B * T, 1))


# speedup vs baseline: 1.9233x; 1.9233x over previous
import jax
import jax.numpy as jnp
from jax.experimental import pallas as pl
from jax.experimental.pallas import tpu as pltpu

_NCHUNK = 2


def _vq_body(x_ref, w_ref, q_ref, idx_ref):
    x = x_ref[0]
    w = w_ref[...]
    K = w.shape[0]
    KC = K // _NCHUNK

    xn = jnp.sum(x * x, axis=0, keepdims=True)
    fiota = jax.lax.broadcasted_iota(jnp.int32, (KC, 1), 0).astype(jnp.float32)

    m = None
    for c in range(_NCHUNK):
        wc = w[c * KC:(c + 1) * KC]
        s2 = jax.lax.dot_general(
            -2.0 * wc, x, (((1,), (0,)), ((), ())),
            preferred_element_type=jnp.float32)
        wn = jnp.sum(wc * wc, axis=1, keepdims=True)
        dist = (wn + xn) + s2
        mc = jnp.min(dist, axis=0, keepdims=True)
        fc = jnp.min(jnp.where(dist == mc, fiota + float(c * KC), float(K)),
                     axis=0, keepdims=True)
        if m is None:
            m, fidx = mc, fc
        else:
            take = mc < m
            m = jnp.where(take, mc, m)
            fidx = jnp.where(take, fc, fidx)

    q = None
    for c in range(_NCHUNK):
        oh = jnp.where(fiota + float(c * KC) == fidx, 1.0, 0.0)
        qc = jax.lax.dot_general(
            w[c * KC:(c + 1) * KC], oh, (((0,), (0,)), ((), ())),
            preferred_element_type=jnp.float32)
        q = qc if q is None else q + qc

    q_ref[0] = x + (q - x)
    idx_ref[0] = fidx.astype(jnp.int32)


_TT = 1024


def kernel(inputs, W):
    B, D, T = inputs.shape
    K = W.shape[0]
    nt = T // _TT
    q, idx = pl.pallas_call(
        _vq_body,
        grid=(B, nt),
        in_specs=[
            pl.BlockSpec((1, D, _TT), lambda b, j: (b, 0, j)),
            pl.BlockSpec((K, D), lambda b, j: (0, 0)),
        ],
        out_specs=[
            pl.BlockSpec((1, D, _TT), lambda b, j: (b, 0, j)),
            pl.BlockSpec((1, 1, _TT), lambda b, j: (b, 0, j)),
        ],
        out_shape=[
            jax.ShapeDtypeStruct((B, D, T), jnp.float32),
            jax.ShapeDtypeStruct((B, 1, T), jnp.int32),
        ],
        compiler_params=pltpu.CompilerParams(
            dimension_semantics=("parallel", "parallel")),
    )(inputs, W)
    return (q, idx.reshape(B * T, 1))
